# baseline (speedup 1.0000x reference)
import functools

import jax
import jax.numpy as jnp
from jax import lax
from jax.experimental import pallas as pl
from jax.experimental.pallas import tpu as pltpu
from jax.experimental.pallas import tpu_sc as plsc

N_NODES = 10000
N_EDGES = 320000
D_FEAT = 128

NC = 2
NS = 16
NW = NC * NS
EDGES_PER_TILE = N_EDGES // NW
CHUNK = 64
NFULL = EDGES_PER_TILE // CHUNK
TAIL = EDGES_PER_TILE - NFULL * CHUNK
ROWS_PER_TILE = 640
N_PAD = NS * ROWS_PER_TILE


def _sc_segment_sum(x, edges, zeros):
    mesh = plsc.VectorSubcoreMesh(core_axis_name="c", subcore_axis_name="s")

    @functools.partial(
        pl.kernel,
        mesh=mesh,
        out_type=jax.ShapeDtypeStruct((NC, N_PAD, D_FEAT), jnp.float32),
        scratch_types=[
            pltpu.VMEM_SHARED((N_PAD, D_FEAT), jnp.float32),
            pltpu.VMEM((EDGES_PER_TILE,), jnp.int32),
            pltpu.VMEM((CHUNK,), jnp.int32),
            pltpu.VMEM((CHUNK,), jnp.int32),
            pltpu.VMEM((CHUNK,), jnp.int32),
            pltpu.VMEM((TAIL,), jnp.int32),
            pltpu.VMEM((CHUNK, D_FEAT), jnp.float32),
            pltpu.VMEM((CHUNK, D_FEAT), jnp.float32),
            pltpu.VMEM((CHUNK, D_FEAT), jnp.float32),
            pltpu.SemaphoreType.DMA,
            pltpu.SemaphoreType.DMA,
            pltpu.SemaphoreType.DMA,
            pltpu.SemaphoreType.DMA,
            pltpu.SemaphoreType.DMA,
            pltpu.SemaphoreType.DMA,
            pltpu.SemaphoreType.DMA,
        ],
    )
    def k(x_hbm, edges_hbm, zeros_hbm, out_hbm, acc_sh, sidx, dd0, dd1, dd2,
          ddt, r0, r1, r2, semi, g0, g1, g2, d0, d1, d2):
        cid = lax.axis_index("c")
        sid = lax.axis_index("s")
        wid = cid * NS + sid
        slab = pl.ds(sid * ROWS_PER_TILE, ROWS_PER_TILE)
        ebase = wid * EDGES_PER_TILE

        def didx_c(i):
            return edges_hbm.at[pl.ds(N_EDGES + ebase + i * CHUNK, CHUNK)]

        def sidx_c(i):
            return sidx.at[pl.ds(i * CHUNK, CHUNK)]

        pltpu.async_copy(didx_c(0), dd0, d0)
        pltpu.async_copy(didx_c(1), dd1, d1)
        pltpu.async_copy(didx_c(2), dd2, d2)
        pltpu.async_copy(edges_hbm.at[pl.ds(ebase, EDGES_PER_TILE)], sidx,
                         semi)
        pltpu.make_async_copy(edges_hbm.at[pl.ds(ebase, EDGES_PER_TILE)],
                              sidx, semi).wait()

        pltpu.async_copy(x_hbm.at[sidx_c(0)], r0, g0)
        pltpu.async_copy(x_hbm.at[sidx_c(1)], r1, g1)
        pltpu.async_copy(x_hbm.at[sidx_c(2)], r2, g2)
        pltpu.sync_copy(zeros_hbm, acc_sh.at[slab])
        plsc.subcore_barrier()

        bufs = ((r0, dd0, g0, d0), (r1, dd1, g1, d1), (r2, dd2, g2, d2))

        def stage(c, rb, ddb, gb, db, launch):
            pltpu.make_async_copy(x_hbm.at[sidx_c(c)], rb, gb).wait()
            pltpu.make_async_copy(didx_c(c), ddb, db).wait()
            pltpu.sync_copy(rb, acc_sh.at[ddb], add=True)
            if launch:
                pltpu.async_copy(didx_c(c + 3), ddb, db)
                pltpu.async_copy(x_hbm.at[sidx_c(c + 3)], rb, gb)

        def body(t, carry):
            i = 3 * t
            for b, (rb, ddb, gb, db) in enumerate(bufs):
                stage(i + b, rb, ddb, gb, db, True)
            return carry

        lax.fori_loop(0, NFULL // 3 - 1, body, 0)
        stage(NFULL - 3, r0, dd0, g0, d0, False)
        tail_off = ebase + NFULL * CHUNK
        pltpu.async_copy(edges_hbm.at[pl.ds(N_EDGES + tail_off, TAIL)],
                         ddt, d0)
        pltpu.async_copy(x_hbm.at[sidx.at[pl.ds(NFULL * CHUNK, TAIL)]],
                         r0.at[pl.ds(0, TAIL)], g0)
        stage(NFULL - 2, r1, dd1, g1, d1, False)
        stage(NFULL - 1, r2, dd2, g2, d2, False)
        pltpu.make_async_copy(x_hbm.at[sidx.at[pl.ds(NFULL * CHUNK, TAIL)]],
                              r0.at[pl.ds(0, TAIL)], g0).wait()
        pltpu.make_async_copy(edges_hbm.at[pl.ds(N_EDGES + tail_off, TAIL)],
                              ddt, d0).wait()
        pltpu.sync_copy(r0.at[pl.ds(0, TAIL)], acc_sh.at[ddt], add=True)
        plsc.subcore_barrier()
        pltpu.sync_copy(acc_sh.at[slab], out_hbm.at[cid, slab])

    return k(x, edges, zeros)


BLK_ROWS = 1000


def _tc_xs(x, Wst):
    def body(x_ref, wst_ref, o_ref):
        o_ref[...] = jnp.dot(x_ref[...], wst_ref[...],
                             preferred_element_type=jnp.float32)

    return pl.pallas_call(
        body,
        grid=(N_NODES // BLK_ROWS,),
        in_specs=[
            pl.BlockSpec((BLK_ROWS, D_FEAT), lambda i: (i, 0)),
            pl.BlockSpec((D_FEAT, D_FEAT), lambda i: (0, 0)),
        ],
        out_specs=pl.BlockSpec((BLK_ROWS, D_FEAT), lambda i: (i, 0)),
        out_shape=jax.ShapeDtypeStruct((N_NODES, D_FEAT), jnp.float32),
    )(x, Wst)


def _tc_final(xs, partials, Wnt):
    def body(xs_ref, p_ref, wnt_ref, o_ref):
        agg = p_ref[0] + p_ref[1]
        z = xs_ref[...] + jnp.dot(agg, wnt_ref[...],
                                  preferred_element_type=jnp.float32)
        o_ref[...] = 0.5 * z * (1.0 + lax.erf(z * 0.7071067811865476))

    return pl.pallas_call(
        body,
        grid=(N_NODES // BLK_ROWS,),
        in_specs=[
            pl.BlockSpec((BLK_ROWS, D_FEAT), lambda i: (i, 0)),
            pl.BlockSpec((NC, BLK_ROWS, D_FEAT), lambda i: (0, i, 0)),
            pl.BlockSpec((D_FEAT, D_FEAT), lambda i: (0, 0)),
        ],
        out_specs=pl.BlockSpec((BLK_ROWS, D_FEAT), lambda i: (i, 0)),
        out_shape=jax.ShapeDtypeStruct((N_NODES, D_FEAT), jnp.float32),
    )(xs, partials, Wnt)


def kernel(x, edge_index, W_s, W_n):
    edges = edge_index.astype(jnp.int32).reshape(2 * N_EDGES)
    zeros = jnp.zeros((ROWS_PER_TILE, D_FEAT), jnp.float32)
    xs = _tc_xs(x, W_s.T)
    partials = _sc_segment_sum(x, edges, zeros)
    return _tc_final(xs, partials, W_n.T)


# speedup vs baseline: 1.0420x; 1.0420x over previous
import functools

import jax
import jax.numpy as jnp
from jax import lax
from jax.experimental import pallas as pl
from jax.experimental.pallas import tpu as pltpu
from jax.experimental.pallas import tpu_sc as plsc

N_NODES = 10000
N_EDGES = 320000
D_FEAT = 128

NC = 2
NS = 16
NW = NC * NS
EDGES_PER_TILE = N_EDGES // NW
CHUNK = 48
NFULL = EDGES_PER_TILE // CHUNK
TAIL = EDGES_PER_TILE - NFULL * CHUNK
ROWS_PER_TILE = 640
N_PAD = NS * ROWS_PER_TILE


def _sc_segment_sum(x, edges, zeros):
    mesh = plsc.VectorSubcoreMesh(core_axis_name="c", subcore_axis_name="s")

    @functools.partial(
        pl.kernel,
        mesh=mesh,
        out_type=jax.ShapeDtypeStruct((NC, N_PAD, D_FEAT), jnp.float32),
        scratch_types=[
            pltpu.VMEM_SHARED((N_PAD, D_FEAT), jnp.float32),
            pltpu.VMEM((EDGES_PER_TILE,), jnp.int32),
            pltpu.VMEM((CHUNK,), jnp.int32),
            pltpu.VMEM((CHUNK,), jnp.int32),
            pltpu.VMEM((CHUNK,), jnp.int32),
            pltpu.VMEM((CHUNK,), jnp.int32),
            pltpu.VMEM((TAIL,), jnp.int32),
            pltpu.VMEM((CHUNK, D_FEAT), jnp.float32),
            pltpu.VMEM((CHUNK, D_FEAT), jnp.float32),
            pltpu.VMEM((CHUNK, D_FEAT), jnp.float32),
            pltpu.VMEM((CHUNK, D_FEAT), jnp.float32),
            pltpu.SemaphoreType.DMA,
            pltpu.SemaphoreType.DMA,
            pltpu.SemaphoreType.DMA,
            pltpu.SemaphoreType.DMA,
            pltpu.SemaphoreType.DMA,
            pltpu.SemaphoreType.DMA,
            pltpu.SemaphoreType.DMA,
            pltpu.SemaphoreType.DMA,
            pltpu.SemaphoreType.DMA,
        ],
    )
    def k(x_hbm, edges_hbm, zeros_hbm, out_hbm, acc_sh, sidx, dd0, dd1, dd2,
          dd3, ddt, r0, r1, r2, r3, semi, g0, g1, g2, g3, d0, d1, d2, d3):
        cid = lax.axis_index("c")
        sid = lax.axis_index("s")
        wid = cid * NS + sid
        slab = pl.ds(sid * ROWS_PER_TILE, ROWS_PER_TILE)
        ebase = wid * EDGES_PER_TILE

        def didx_c(i):
            return edges_hbm.at[pl.ds(N_EDGES + ebase + i * CHUNK, CHUNK)]

        def sidx_c(i):
            return sidx.at[pl.ds(i * CHUNK, CHUNK)]

        pltpu.async_copy(didx_c(0), dd0, d0)
        pltpu.async_copy(didx_c(1), dd1, d1)
        pltpu.async_copy(didx_c(2), dd2, d2)
        pltpu.async_copy(didx_c(3), dd3, d3)
        pltpu.async_copy(edges_hbm.at[pl.ds(ebase, EDGES_PER_TILE)], sidx,
                         semi)
        pltpu.sync_copy(zeros_hbm, acc_sh.at[slab])
        pltpu.make_async_copy(edges_hbm.at[pl.ds(ebase, EDGES_PER_TILE)],
                              sidx, semi).wait()

        pltpu.async_copy(x_hbm.at[sidx_c(0)], r0, g0)
        pltpu.async_copy(x_hbm.at[sidx_c(1)], r1, g1)
        pltpu.async_copy(x_hbm.at[sidx_c(2)], r2, g2)
        pltpu.async_copy(x_hbm.at[sidx_c(3)], r3, g3)
        plsc.subcore_barrier()

        bufs = ((r0, dd0, g0, d0), (r1, dd1, g1, d1), (r2, dd2, g2, d2),
                (r3, dd3, g3, d3))

        def stage(c, rb, ddb, gb, db, launch):
            pltpu.make_async_copy(x_hbm.at[sidx_c(c)], rb, gb).wait()
            pltpu.make_async_copy(didx_c(c), ddb, db).wait()
            pltpu.sync_copy(rb, acc_sh.at[ddb], add=True)
            if launch:
                pltpu.async_copy(didx_c(c + 4), ddb, db)
                pltpu.async_copy(x_hbm.at[sidx_c(c + 4)], rb, gb)

        def body(t, carry):
            i = 4 * t
            for b, (rb, ddb, gb, db) in enumerate(bufs):
                stage(i + b, rb, ddb, gb, db, True)
            return carry

        lax.fori_loop(0, NFULL // 4 - 1, body, 0)
        stage(NFULL - 4, r0, dd0, g0, d0, False)
        tail_off = ebase + NFULL * CHUNK
        pltpu.async_copy(edges_hbm.at[pl.ds(N_EDGES + tail_off, TAIL)],
                         ddt, d0)
        pltpu.async_copy(x_hbm.at[sidx.at[pl.ds(NFULL * CHUNK, TAIL)]],
                         r0.at[pl.ds(0, TAIL)], g0)
        stage(NFULL - 3, r1, dd1, g1, d1, False)
        stage(NFULL - 2, r2, dd2, g2, d2, False)
        stage(NFULL - 1, r3, dd3, g3, d3, False)
        pltpu.make_async_copy(x_hbm.at[sidx.at[pl.ds(NFULL * CHUNK, TAIL)]],
                              r0.at[pl.ds(0, TAIL)], g0).wait()
        pltpu.make_async_copy(edges_hbm.at[pl.ds(N_EDGES + tail_off, TAIL)],
                              ddt, d0).wait()
        pltpu.sync_copy(r0.at[pl.ds(0, TAIL)], acc_sh.at[ddt], add=True)
        plsc.subcore_barrier()
        pltpu.sync_copy(acc_sh.at[slab], out_hbm.at[cid, slab])

    return k(x, edges, zeros)


BLK_ROWS = 1000


def _tc_xs(x, Wst):
    def body(x_ref, wst_ref, o_ref):
        o_ref[...] = jnp.dot(x_ref[...], wst_ref[...],
                             preferred_element_type=jnp.float32)

    return pl.pallas_call(
        body,
        grid=(N_NODES // BLK_ROWS,),
        in_specs=[
            pl.BlockSpec((BLK_ROWS, D_FEAT), lambda i: (i, 0)),
            pl.BlockSpec((D_FEAT, D_FEAT), lambda i: (0, 0)),
        ],
        out_specs=pl.BlockSpec((BLK_ROWS, D_FEAT), lambda i: (i, 0)),
        out_shape=jax.ShapeDtypeStruct((N_NODES, D_FEAT), jnp.float32),
    )(x, Wst)


def _tc_final(xs, partials, Wnt):
    def body(xs_ref, p_ref, wnt_ref, o_ref):
        agg = p_ref[0] + p_ref[1]
        z = xs_ref[...] + jnp.dot(agg, wnt_ref[...],
                                  preferred_element_type=jnp.float32)
        o_ref[...] = 0.5 * z * (1.0 + lax.erf(z * 0.7071067811865476))

    return pl.pallas_call(
        body,
        grid=(N_NODES // BLK_ROWS,),
        in_specs=[
            pl.BlockSpec((BLK_ROWS, D_FEAT), lambda i: (i, 0)),
            pl.BlockSpec((NC, BLK_ROWS, D_FEAT), lambda i: (0, i, 0)),
            pl.BlockSpec((D_FEAT, D_FEAT), lambda i: (0, 0)),
        ],
        out_specs=pl.BlockSpec((BLK_ROWS, D_FEAT), lambda i: (i, 0)),
        out_shape=jax.ShapeDtypeStruct((N_NODES, D_FEAT), jnp.float32),
    )(xs, partials, Wnt)


def kernel(x, edge_index, W_s, W_n):
    edges = edge_index.astype(jnp.int32).reshape(2 * N_EDGES)
    zeros = jnp.zeros((ROWS_PER_TILE, D_FEAT), jnp.float32)
    xs = _tc_xs(x, W_s.T)
    partials = _sc_segment_sum(x, edges, zeros)
    return _tc_final(xs, partials, W_n.T)


# speedup vs baseline: 1.0562x; 1.0136x over previous
import functools

import jax
import jax.numpy as jnp
from jax import lax
from jax.experimental import pallas as pl
from jax.experimental.pallas import tpu as pltpu
from jax.experimental.pallas import tpu_sc as plsc

N_NODES = 10000
N_EDGES = 320000
D_FEAT = 128

NC = 2
NS = 16
NW = NC * NS
EDGES_PER_TILE = N_EDGES // NW
CHUNK = 40
NFULL = EDGES_PER_TILE // CHUNK
ROWS_PER_TILE = 640
N_PAD = NS * ROWS_PER_TILE


def _sc_segment_sum(x, edges, zeros):
    mesh = plsc.VectorSubcoreMesh(core_axis_name="c", subcore_axis_name="s")

    @functools.partial(
        pl.kernel,
        mesh=mesh,
        out_type=jax.ShapeDtypeStruct((NC, N_PAD, D_FEAT), jnp.float32),
        scratch_types=[
            pltpu.VMEM_SHARED((N_PAD, D_FEAT), jnp.float32),
            pltpu.VMEM((EDGES_PER_TILE,), jnp.int32),
            pltpu.VMEM((CHUNK,), jnp.int32),
            pltpu.VMEM((CHUNK,), jnp.int32),
            pltpu.VMEM((CHUNK,), jnp.int32),
            pltpu.VMEM((CHUNK,), jnp.int32),
            pltpu.VMEM((CHUNK,), jnp.int32),
            pltpu.VMEM((CHUNK, D_FEAT), jnp.float32),
            pltpu.VMEM((CHUNK, D_FEAT), jnp.float32),
            pltpu.VMEM((CHUNK, D_FEAT), jnp.float32),
            pltpu.VMEM((CHUNK, D_FEAT), jnp.float32),
            pltpu.VMEM((CHUNK, D_FEAT), jnp.float32),
            pltpu.SemaphoreType.DMA,
            pltpu.SemaphoreType.DMA,
            pltpu.SemaphoreType.DMA,
            pltpu.SemaphoreType.DMA,
            pltpu.SemaphoreType.DMA,
            pltpu.SemaphoreType.DMA,
            pltpu.SemaphoreType.DMA,
            pltpu.SemaphoreType.DMA,
            pltpu.SemaphoreType.DMA,
            pltpu.SemaphoreType.DMA,
            pltpu.SemaphoreType.DMA,
        ],
    )
    def k(x_hbm, edges_hbm, zeros_hbm, out_hbm, acc_sh, sidx, dd0, dd1, dd2,
          dd3, dd4, r0, r1, r2, r3, r4, semi, g0, g1, g2, g3, g4,
          d0, d1, d2, d3, d4):
        cid = lax.axis_index("c")
        sid = lax.axis_index("s")
        wid = cid * NS + sid
        slab = pl.ds(sid * ROWS_PER_TILE, ROWS_PER_TILE)
        ebase = wid * EDGES_PER_TILE

        def didx_c(i):
            return edges_hbm.at[pl.ds(N_EDGES + ebase + i * CHUNK, CHUNK)]

        def sidx_c(i):
            return sidx.at[pl.ds(i * CHUNK, CHUNK)]

        pltpu.async_copy(didx_c(0), dd0, d0)
        pltpu.async_copy(didx_c(1), dd1, d1)
        pltpu.async_copy(didx_c(2), dd2, d2)
        pltpu.async_copy(didx_c(3), dd3, d3)
        pltpu.async_copy(didx_c(4), dd4, d4)
        pltpu.async_copy(edges_hbm.at[pl.ds(ebase, EDGES_PER_TILE)], sidx,
                         semi)
        pltpu.sync_copy(zeros_hbm, acc_sh.at[slab])
        pltpu.make_async_copy(edges_hbm.at[pl.ds(ebase, EDGES_PER_TILE)],
                              sidx, semi).wait()

        pltpu.async_copy(x_hbm.at[sidx_c(0)], r0, g0)
        pltpu.async_copy(x_hbm.at[sidx_c(1)], r1, g1)
        pltpu.async_copy(x_hbm.at[sidx_c(2)], r2, g2)
        pltpu.async_copy(x_hbm.at[sidx_c(3)], r3, g3)
        pltpu.async_copy(x_hbm.at[sidx_c(4)], r4, g4)
        plsc.subcore_barrier()

        bufs = ((r0, dd0, g0, d0), (r1, dd1, g1, d1), (r2, dd2, g2, d2),
                (r3, dd3, g3, d3), (r4, dd4, g4, d4))

        def stage(c, rb, ddb, gb, db, launch):
            pltpu.make_async_copy(x_hbm.at[sidx_c(c)], rb, gb).wait()
            pltpu.make_async_copy(didx_c(c), ddb, db).wait()
            pltpu.sync_copy(rb, acc_sh.at[ddb], add=True)
            if launch:
                pltpu.async_copy(didx_c(c + 5), ddb, db)
                pltpu.async_copy(x_hbm.at[sidx_c(c + 5)], rb, gb)

        def body(t, carry):
            i = 5 * t
            for b, (rb, ddb, gb, db) in enumerate(bufs):
                stage(i + b, rb, ddb, gb, db, True)
            return carry

        lax.fori_loop(0, NFULL // 5 - 1, body, 0)
        stage(NFULL - 5, r0, dd0, g0, d0, False)
        stage(NFULL - 4, r1, dd1, g1, d1, False)
        stage(NFULL - 3, r2, dd2, g2, d2, False)
        stage(NFULL - 2, r3, dd3, g3, d3, False)
        stage(NFULL - 1, r4, dd4, g4, d4, False)
        plsc.subcore_barrier()
        pltpu.sync_copy(acc_sh.at[slab], out_hbm.at[cid, slab])

    return k(x, edges, zeros)


BLK_ROWS = 1000


def _tc_xs(x, Wst):
    def body(x_ref, wst_ref, o_ref):
        o_ref[...] = jnp.dot(x_ref[...], wst_ref[...],
                             preferred_element_type=jnp.float32)

    return pl.pallas_call(
        body,
        grid=(N_NODES // BLK_ROWS,),
        in_specs=[
            pl.BlockSpec((BLK_ROWS, D_FEAT), lambda i: (i, 0)),
            pl.BlockSpec((D_FEAT, D_FEAT), lambda i: (0, 0)),
        ],
        out_specs=pl.BlockSpec((BLK_ROWS, D_FEAT), lambda i: (i, 0)),
        out_shape=jax.ShapeDtypeStruct((N_NODES, D_FEAT), jnp.float32),
    )(x, Wst)


def _tc_final(xs, partials, Wnt):
    def body(xs_ref, p_ref, wnt_ref, o_ref):
        agg = p_ref[0] + p_ref[1]
        z = xs_ref[...] + jnp.dot(agg, wnt_ref[...],
                                  preferred_element_type=jnp.float32)
        o_ref[...] = 0.5 * z * (1.0 + lax.erf(z * 0.7071067811865476))

    return pl.pallas_call(
        body,
        grid=(N_NODES // BLK_ROWS,),
        in_specs=[
            pl.BlockSpec((BLK_ROWS, D_FEAT), lambda i: (i, 0)),
            pl.BlockSpec((NC, BLK_ROWS, D_FEAT), lambda i: (0, i, 0)),
            pl.BlockSpec((D_FEAT, D_FEAT), lambda i: (0, 0)),
        ],
        out_specs=pl.BlockSpec((BLK_ROWS, D_FEAT), lambda i: (i, 0)),
        out_shape=jax.ShapeDtypeStruct((N_NODES, D_FEAT), jnp.float32),
    )(xs, partials, Wnt)


def kernel(x, edge_index, W_s, W_n):
    edges = edge_index.astype(jnp.int32).reshape(2 * N_EDGES)
    zeros = jnp.zeros((ROWS_PER_TILE, D_FEAT), jnp.float32)
    xs = _tc_xs(x, W_s.T)
    partials = _sc_segment_sum(x, edges, zeros)
    return _tc_final(xs, partials, W_n.T)
